# baseline (speedup 1.0000x reference)
import functools
import math

import jax
import jax.numpy as jnp
from jax import lax
from jax.experimental import pallas as pl
from jax.experimental.pallas import tpu as pltpu
from jax.experimental.pallas import tpu_sc as plsc

VOCAB_SIZE = 1000000
D_MODEL = 64
BATCH = 4096
SEQ_LEN = 200
SCALE = math.sqrt(D_MODEL)

NC = 2
NS = 16
NW = NC * NS
N_TOK = BATCH * SEQ_LEN
B_PER_W = N_TOK // NW
CHUNK = 1024
N_CHUNKS = B_PER_W // CHUNK


def _emb_kernel(x_hbm, lut_hbm, out_hbm, idx_v, rows_v, sem):
    wid = lax.axis_index("s") * NC + lax.axis_index("c")
    base = wid * B_PER_W

    def chunk_body(g, carry):
        start = base + g * CHUNK
        pltpu.sync_copy(x_hbm.at[pl.ds(start, CHUNK)], idx_v)
        pltpu.async_copy(lut_hbm.at[idx_v], rows_v, sem).wait()

        def scale_row(r, c):
            for j in range(D_MODEL // 16):
                sl = pl.ds(j * 16, 16)
                rows_v[r, sl] = rows_v[r, sl] * SCALE
            return c

        lax.fori_loop(0, CHUNK, scale_row, 0)
        pltpu.sync_copy(rows_v, out_hbm.at[pl.ds(start, CHUNK)])
        return carry

    lax.fori_loop(0, N_CHUNKS, chunk_body, 0)


@jax.jit
def kernel(x, lut):
    x_flat = x.reshape(-1).astype(jnp.int32)
    mesh = plsc.VectorSubcoreMesh(core_axis_name="c", subcore_axis_name="s")
    out = pl.kernel(
        _emb_kernel,
        out_type=jax.ShapeDtypeStruct((N_TOK, D_MODEL), jnp.float32),
        mesh=mesh,
        scratch_types=[
            pltpu.VMEM((CHUNK,), jnp.int32),
            pltpu.VMEM((CHUNK, D_MODEL), jnp.float32),
            pltpu.SemaphoreType.DMA,
        ],
        compiler_params=pltpu.CompilerParams(use_tc_tiling_on_sc=False),
    )(x_flat, lut)
    return out.reshape(BATCH, SEQ_LEN, D_MODEL)


# speedup vs baseline: 1.1135x; 1.1135x over previous
import math

import jax
import jax.numpy as jnp
from jax import lax
from jax.experimental import pallas as pl
from jax.experimental.pallas import tpu as pltpu
from jax.experimental.pallas import tpu_sc as plsc

VOCAB_SIZE = 1000000
D_MODEL = 64
BATCH = 4096
SEQ_LEN = 200
SCALE = math.sqrt(D_MODEL)

NC = 2
NS = 16
NW = NC * NS
N_TOK = BATCH * SEQ_LEN
B_PER_W = N_TOK // NW
CHUNK = 320
N_CHUNKS = B_PER_W // CHUNK


def _emb_kernel(x_hbm, lut_hbm, out_hbm, idx_all,
                rows_in0, rows_in1, out_b0, out_b1,
                sem_g0, sem_g1, sem_o0, sem_o1):
    wid = lax.axis_index("s") * NC + lax.axis_index("c")
    base = wid * B_PER_W
    rows_in = (rows_in0, rows_in1)
    out_b = (out_b0, out_b1)
    sem_g = (sem_g0, sem_g1)
    sem_o = (sem_o0, sem_o1)

    pltpu.sync_copy(x_hbm.at[pl.ds(base, B_PER_W)], idx_all)

    def gather_desc(g, b):
        return pltpu.make_async_copy(
            lut_hbm.at[idx_all.at[pl.ds(g * CHUNK, CHUNK)]], rows_in[b],
            sem_g[b])

    def wb_desc(g, b):
        return pltpu.make_async_copy(
            out_b[b], out_hbm.at[pl.ds(base + g * CHUNK, CHUNK)], sem_o[b])

    gather_desc(0, 0).start()
    gather_desc(1, 1).start()

    @pl.loop(0, N_CHUNKS, step=2)
    def chunk_loop(g0):
        for b in range(2):
            g = g0 + b
            gather_desc(g, b).wait()

            @pl.when(g >= 2)
            def _():
                wb_desc(g - 2, b).wait()

            @plsc.parallel_loop(0, CHUNK, unroll=8)
            def scale_row(r):
                for j in range(D_MODEL // 16):
                    sl = pl.ds(j * 16, 16)
                    out_b[b][r, sl] = rows_in[b][r, sl] * SCALE

            @pl.when(g + 2 < N_CHUNKS)
            def _():
                gather_desc(g + 2, b).start()

            wb_desc(g, b).start()

    for b in range(2):
        wb_desc(N_CHUNKS - 2 + b, b).wait()


@jax.jit
def kernel(x, lut):
    x_flat = x.reshape(-1).astype(jnp.int32)
    mesh = plsc.VectorSubcoreMesh(core_axis_name="c", subcore_axis_name="s")
    out = pl.kernel(
        _emb_kernel,
        out_type=jax.ShapeDtypeStruct((N_TOK, D_MODEL), jnp.float32),
        mesh=mesh,
        scratch_types=[
            pltpu.VMEM((B_PER_W,), jnp.int32),
            pltpu.VMEM((CHUNK, D_MODEL), jnp.float32),
            pltpu.VMEM((CHUNK, D_MODEL), jnp.float32),
            pltpu.VMEM((CHUNK, D_MODEL), jnp.float32),
            pltpu.VMEM((CHUNK, D_MODEL), jnp.float32),
            pltpu.SemaphoreType.DMA,
            pltpu.SemaphoreType.DMA,
            pltpu.SemaphoreType.DMA,
            pltpu.SemaphoreType.DMA,
        ],
        compiler_params=pltpu.CompilerParams(use_tc_tiling_on_sc=False),
    )(x_flat, lut)
    return out.reshape(BATCH, SEQ_LEN, D_MODEL)
